# baseline (speedup 1.0000x reference)
import functools

import jax
import jax.numpy as jnp
from jax.experimental import pallas as pl


def _gcn_body(adj_ref, x_ref, w1_ref, b1_ref, w2_ref, b2_ref, out_ref, *, n):
    adj = adj_ref[...]
    gb = adj.shape[0]
    ii = jax.lax.broadcasted_iota(jnp.int32, (n, n), 0)
    jj = jax.lax.broadcasted_iota(jnp.int32, (n, n), 1)
    mask = ((ii < jj)[None, :, :] & (adj != 0)) | (ii == jj)[None, :, :]
    a = mask.astype(jnp.bfloat16)
    ones_c = jnp.ones((n, 8), jnp.bfloat16)
    ones_r = jnp.ones((gb, n, 8), jnp.bfloat16)
    deg_col = jax.lax.dot_general(
        a, ones_c, (((1,), (0,)), ((), ())),
        preferred_element_type=jnp.float32)
    deg_row = jax.lax.dot_general(
        ones_r, a, (((1,), (1,)), ((0,), (0,))),
        preferred_element_type=jnp.float32)
    dinv_c = jax.lax.rsqrt(deg_col[:, :, :1])
    dinv_r = jax.lax.rsqrt(deg_row[:, :1, :])
    an = jnp.where(mask, dinv_c * dinv_r, 0.0).astype(jnp.bfloat16)

    def conv(xin, w_ref):
        xw = jax.lax.dot_general(
            xin, w_ref[...].astype(jnp.bfloat16), (((2,), (0,)), ((), ())),
            preferred_element_type=jnp.float32,
        )
        return jax.lax.dot_general(
            an, xw.astype(jnp.bfloat16), (((1,), (1,)), ((0,), (0,))),
            preferred_element_type=jnp.float32,
        )

    h = jax.nn.relu(conv(x_ref[...].astype(jnp.bfloat16), w1_ref)
                    + b1_ref[...][None, :, :])
    out2 = conv(h.astype(jnp.bfloat16), w2_ref)
    out_ref[:, 0, :] = jnp.max(out2, axis=1) + b2_ref[...][0, :]


def kernel(adj_mat, v_feature, W1, b1, W2, b2):
    B, N, _ = adj_mat.shape
    d_in, d_hidden = W1.shape
    d_out = W2.shape[1]
    GB = 32
    adj = adj_mat.astype(jnp.int32)
    b1r = b1.reshape(1, d_hidden).astype(jnp.float32)
    b2r = b2.reshape(1, d_out).astype(jnp.float32)
    out = pl.pallas_call(
        functools.partial(_gcn_body, n=N),
        grid=(B // GB,),
        in_specs=[
            pl.BlockSpec((GB, N, N), lambda i: (i, 0, 0)),
            pl.BlockSpec((GB, N, d_in), lambda i: (i, 0, 0)),
            pl.BlockSpec((d_in, d_hidden), lambda i: (0, 0)),
            pl.BlockSpec((1, d_hidden), lambda i: (0, 0)),
            pl.BlockSpec((d_hidden, d_out), lambda i: (0, 0)),
            pl.BlockSpec((1, d_out), lambda i: (0, 0)),
        ],
        out_specs=pl.BlockSpec((GB, 1, d_out), lambda i: (i, 0, 0)),
        out_shape=jax.ShapeDtypeStruct((B, 1, d_out), jnp.float32),
    )(adj, v_feature.astype(jnp.float32), W1, b1r, W2, b2r)
    return out


# speedup vs baseline: 1.1543x; 1.1543x over previous
import functools

import jax
import jax.numpy as jnp
from jax.experimental import pallas as pl


def _gcn_body(adj_ref, x_ref, w1_ref, b1_ref, w2_ref, b2_ref, out_ref, *, n):
    adj = adj_ref[...]
    ii = jax.lax.broadcasted_iota(jnp.int32, (n, n), 0)
    jj = jax.lax.broadcasted_iota(jnp.int32, (n, n), 1)
    upper = (ii < jj).astype(jnp.int32)
    eye = (ii == jj).astype(jnp.int32)
    a_int = (adj & upper[None, :, :]) | eye[None, :, :]
    a_f = a_int.astype(jnp.float32)
    deg = jnp.sum(a_f, axis=1)
    dinv = jax.lax.rsqrt(deg)
    an = (a_f * (dinv[:, :, None] * dinv[:, None, :])).astype(jnp.bfloat16)

    def conv(xin, w_ref):
        xw = jax.lax.dot_general(
            xin, w_ref[...].astype(jnp.bfloat16), (((2,), (0,)), ((), ())),
            preferred_element_type=jnp.float32,
        )
        return jax.lax.dot_general(
            an, xw.astype(jnp.bfloat16), (((1,), (1,)), ((0,), (0,))),
            preferred_element_type=jnp.float32,
        )

    h = jax.nn.relu(conv(x_ref[...].astype(jnp.bfloat16), w1_ref)
                    + b1_ref[...][None, :, :])
    out2 = conv(h.astype(jnp.bfloat16), w2_ref)
    out_ref[:, 0, :] = jnp.max(out2, axis=1) + b2_ref[...][0, :]


def kernel(adj_mat, v_feature, W1, b1, W2, b2):
    B, N, _ = adj_mat.shape
    d_in, d_hidden = W1.shape
    d_out = W2.shape[1]
    GB = 32
    adj = adj_mat.astype(jnp.int32)
    b1r = b1.reshape(1, d_hidden).astype(jnp.float32)
    b2r = b2.reshape(1, d_out).astype(jnp.float32)
    out = pl.pallas_call(
        functools.partial(_gcn_body, n=N),
        grid=(B // GB,),
        in_specs=[
            pl.BlockSpec((GB, N, N), lambda i: (i, 0, 0)),
            pl.BlockSpec((GB, N, d_in), lambda i: (i, 0, 0)),
            pl.BlockSpec((d_in, d_hidden), lambda i: (0, 0)),
            pl.BlockSpec((1, d_hidden), lambda i: (0, 0)),
            pl.BlockSpec((d_hidden, d_out), lambda i: (0, 0)),
            pl.BlockSpec((1, d_out), lambda i: (0, 0)),
        ],
        out_specs=pl.BlockSpec((GB, 1, d_out), lambda i: (i, 0, 0)),
        out_shape=jax.ShapeDtypeStruct((B, 1, d_out), jnp.float32),
    )(adj, v_feature.astype(jnp.float32), W1, b1r, W2, b2r)
    return out


# speedup vs baseline: 1.3145x; 1.1388x over previous
import functools

import jax
import jax.numpy as jnp
from jax.experimental import pallas as pl


def _gcn_body(adj_ref, x_ref, w1_ref, b1_ref, w2_ref, b2_ref, out_ref, *, n):
    adj = adj_ref[...]
    ii = jax.lax.broadcasted_iota(jnp.int32, (n, n), 0)
    jj = jax.lax.broadcasted_iota(jnp.int32, (n, n), 1)
    upper = (ii < jj).astype(jnp.int32)
    eye = (ii == jj).astype(jnp.int32)
    a_int = (adj & upper[None, :, :]) | eye[None, :, :]
    a = a_int.astype(jnp.bfloat16)
    deg = jnp.sum(a_int, axis=1).astype(jnp.float32)
    gb, d = adj.shape[0], w1_ref.shape[1]
    dinvb = jnp.broadcast_to(jax.lax.rsqrt(deg)[:, :, None], (gb, n, d))

    def conv(xin, w_ref):
        xw = jax.lax.dot_general(
            xin, w_ref[...].astype(jnp.bfloat16), (((2,), (0,)), ((), ())),
            preferred_element_type=jnp.float32,
        )
        y = (xw * dinvb).astype(jnp.bfloat16)
        agg = jax.lax.dot_general(
            a, y, (((1,), (1,)), ((0,), (0,))),
            preferred_element_type=jnp.float32,
        )
        return agg * dinvb

    h = jax.nn.relu(conv(x_ref[...].astype(jnp.bfloat16), w1_ref)
                    + b1_ref[...][None, :, :])
    out2 = conv(h.astype(jnp.bfloat16), w2_ref)
    out_ref[:, 0, :] = jnp.max(out2, axis=1) + b2_ref[...][0, :]


def kernel(adj_mat, v_feature, W1, b1, W2, b2):
    B, N, _ = adj_mat.shape
    d_in, d_hidden = W1.shape
    d_out = W2.shape[1]
    GB = 32
    adj = adj_mat.astype(jnp.int32)
    b1r = b1.reshape(1, d_hidden).astype(jnp.float32)
    b2r = b2.reshape(1, d_out).astype(jnp.float32)
    out = pl.pallas_call(
        functools.partial(_gcn_body, n=N),
        grid=(B // GB,),
        in_specs=[
            pl.BlockSpec((GB, N, N), lambda i: (i, 0, 0)),
            pl.BlockSpec((GB, N, d_in), lambda i: (i, 0, 0)),
            pl.BlockSpec((d_in, d_hidden), lambda i: (0, 0)),
            pl.BlockSpec((1, d_hidden), lambda i: (0, 0)),
            pl.BlockSpec((d_hidden, d_out), lambda i: (0, 0)),
            pl.BlockSpec((1, d_out), lambda i: (0, 0)),
        ],
        out_specs=pl.BlockSpec((GB, 1, d_out), lambda i: (i, 0, 0)),
        out_shape=jax.ShapeDtypeStruct((B, 1, d_out), jnp.float32),
    )(adj, v_feature.astype(jnp.float32), W1, b1r, W2, b2r)
    return out
